# baseline (speedup 1.0000x reference)
import jax
import jax.numpy as jnp
import numpy as np
from jax.experimental import pallas as pl
from jax.experimental.pallas import tpu as pltpu

_NB = 64
_SIGMA = 0.1 * (1.0 / _NB)
_EPS = float(np.finfo(np.float32).eps)
_K2 = float(np.log2(np.e) / (_SIGMA * _SIGMA))
_SQK = float(np.sqrt(_K2))


def _fold_extreme(v, op):
    t = v[0:8]
    for s in range(8, v.shape[0], 8):
        t = op(t, v[s:s + 8])
    return op(op(t[:, 0:128], t[:, 128:256]),
              op(t[:, 256:384], t[:, 384:512]))


def _body(x1_ref, x2_ref, mi_ref, mm_ref, hist_ref, mmv_ref):
    p = pl.program_id(0)
    i = pl.program_id(1)
    nc = pl.num_programs(1)

    @pl.when(p == 0)
    def _():
        mn1 = _fold_extreme(x1_ref[0], jnp.minimum)
        mx1 = _fold_extreme(x1_ref[0], jnp.maximum)
        mn2 = _fold_extreme(x2_ref[0], jnp.minimum)
        mx2 = _fold_extreme(x2_ref[0], jnp.maximum)

        @pl.when(i == 0)
        def _():
            mmv_ref[0] = mn1
            mmv_ref[1] = mx1
            mmv_ref[2] = mn2
            mmv_ref[3] = mx2

        @pl.when(i > 0)
        def _():
            mmv_ref[0] = jnp.minimum(mmv_ref[0], mn1)
            mmv_ref[1] = jnp.maximum(mmv_ref[1], mx1)
            mmv_ref[2] = jnp.minimum(mmv_ref[2], mn2)
            mmv_ref[3] = jnp.maximum(mmv_ref[3], mx2)

    @pl.when(p == 1)
    def _():
        @pl.when(i == 0)
        def _():
            mm_ref[0] = jnp.min(mmv_ref[0])
            mm_ref[1] = jnp.max(mmv_ref[1])
            mm_ref[2] = jnp.min(mmv_ref[2])
            mm_ref[3] = jnp.max(mmv_ref[3])

        mn1 = mm_ref[0]
        mx1 = mm_ref[1]
        mn2 = mm_ref[2]
        mx2 = mm_ref[3]
        inv1 = 1.0 / (mx1 - mn1 + _EPS)
        inv2 = 1.0 / (mx2 - mn2 + _EPS)

        rows = x1_ref.shape[1]
        dlt = _SQK / _NB
        for s in range(0, rows, 8):
            a = x1_ref[0, s:s + 8] * (0.5 * inv1)
            b = x2_ref[0, s:s + 8] * (0.5 * inv2)
            z = a + b - 0.5 * (mn1 * inv1 + mn2 * inv2)
            r = (a - b) * 2.0 + (mn2 * inv2 - mn1 * inv1)
            h = (r * r) * (-0.25 * _K2)
            zs = z * _SQK
            e = None
            d = None
            for bi in range(_NB):
                if bi % 16 == 0:
                    cbs = _SQK * ((bi + 0.5) / _NB)
                    zc = zs - cbs
                    e = h - zc * zc
                    d = zc * (2.0 * dlt) - dlt * dlt
                else:
                    e = e + d
                    d = d - 2.0 * dlt * dlt
                w = jnp.exp2(e)
                t = ((w[:, 0:128] + w[:, 128:256])
                     + (w[:, 256:384] + w[:, 384:512]))
                if s == 0:
                    hist_ref[i, bi] = t
                else:
                    hist_ref[i, bi] = hist_ref[i, bi] + t

        @pl.when(i == nc - 1)
        def _():
            total = 4 * 1 * 512 * 512
            part = jnp.sum(hist_ref[...], axis=2)
            hist = jnp.sum(part, axis=-1)
            hist = hist * (1.0 / total)
            ssum = jnp.sum(hist, axis=-1, keepdims=True)
            hist = hist / (ssum + _EPS)
            px = jnp.sum(hist, axis=-1, keepdims=True)
            py = jnp.sum(hist, axis=0, keepdims=True)
            px_py = px * py
            mi = jnp.sum(hist * jnp.log((hist + _EPS) / (px_py + _EPS) + _EPS))
            mi_ref[0] = mi


def kernel(img1, img2):
    n, c, h, w = img1.shape
    nc = n * c
    x1 = img1.reshape(nc, h, w)
    x2 = img2.reshape(nc, h, w)

    mi = pl.pallas_call(
        _body,
        grid=(2, nc),
        in_specs=[
            pl.BlockSpec((1, h, w), lambda p, i: (i, 0, 0)),
            pl.BlockSpec((1, h, w), lambda p, i: (i, 0, 0)),
        ],
        out_specs=pl.BlockSpec(memory_space=pltpu.SMEM),
        out_shape=jax.ShapeDtypeStruct((1,), jnp.float32),
        scratch_shapes=[
            pltpu.SMEM((8,), jnp.float32),
            pltpu.VMEM((nc, _NB, 8, 128), jnp.float32),
            pltpu.VMEM((4, 8, 128), jnp.float32),
        ],
        compiler_params=pltpu.CompilerParams(
            dimension_semantics=("arbitrary", "arbitrary"),
        ),
    )(x1, x2)
    return mi[0]
